# baseline (speedup 1.0000x reference)
import jax
import jax.numpy as jnp
from jax.experimental import pallas as pl
from jax.experimental.pallas import tpu as pltpu


def _topk_epilogue(logits):
    e = logits.shape[-1]
    m = jnp.max(logits, axis=-1, keepdims=True)
    z = jnp.sum(jnp.exp(logits - m), axis=-1, keepdims=True)
    iota = jax.lax.broadcasted_iota(jnp.int32, logits.shape, 1)
    big = jnp.int32(e)
    idx1 = jnp.min(jnp.where(logits == m, iota, big), axis=-1, keepdims=True)
    masked = jnp.where(iota == idx1, -jnp.inf, logits)
    m2 = jnp.max(masked, axis=-1, keepdims=True)
    idx2 = jnp.min(jnp.where(masked == m2, iota, big), axis=-1, keepdims=True)
    s1 = 1.0 / z
    s2 = jnp.exp(m2 - m) / z
    scores = jnp.concatenate([s1, s2], axis=-1)
    indices = jnp.concatenate([idx1, idx2], axis=-1)
    return scores, indices


def _router_kernel(xa_ref, xb_ref, w_ref, s_ref, i_ref):
    w = w_ref[...]
    dn = (((1,), (1,)), ((), ()))
    t = xa_ref.shape[0]
    la = jax.lax.dot_general(xa_ref[...], w, dn,
                             preferred_element_type=jnp.float32)
    sa, ia = _topk_epilogue(la)
    s_ref[:t, :] = sa
    i_ref[:t, :] = ia
    lb = jax.lax.dot_general(xb_ref[...], w, dn,
                             preferred_element_type=jnp.float32)
    sb, ib = _topk_epilogue(lb)
    s_ref[t:, :] = sb
    i_ref[t:, :] = ib


def kernel(hidden_states, W):
    B, S, D = hidden_states.shape
    E = W.shape[0]
    N = B * S
    x = hidden_states.reshape(N, D)
    T = 1024
    G = N // (2 * T)
    scores, indices = pl.pallas_call(
        _router_kernel,
        grid=(G,),
        compiler_params=pltpu.CompilerParams(
            dimension_semantics=("parallel",)),
        in_specs=[
            pl.BlockSpec((T, D), lambda i: (2 * i, 0)),
            pl.BlockSpec((T, D), lambda i: (2 * i + 1, 0)),
            pl.BlockSpec((E, D), lambda i: (0, 0)),
        ],
        out_specs=[
            pl.BlockSpec((2 * T, 2), lambda i: (i, 0)),
            pl.BlockSpec((2 * T, 2), lambda i: (i, 0)),
        ],
        out_shape=[
            jax.ShapeDtypeStruct((N, 2), jnp.float32),
            jax.ShapeDtypeStruct((N, 2), jnp.int32),
        ],
    )(x, x, W)
    return scores.reshape(B, S, 2), indices.reshape(B, S, 2)
